# baseline (speedup 1.0000x reference)
import jax
import jax.numpy as jnp
from jax import lax
from jax.experimental import pallas as pl
from jax.experimental.pallas import tpu as pltpu
from jax.experimental.pallas import tpu_sc as plsc

NUM_RELATIONS = 1000
DIM = 128
BATCH = 4096
L = 16
NW = 32
S = 4
GPT_MAX = 56
MIDROWS = GPT_MAX + 2
SLOTS = GPT_MAX * S
HALF = SLOTS // 2
NBLK = DIM // L


def _compute_group(ht_v, mat_v, out_v, s0):
    def pass_body(half, _):
        sa = s0 + 2 * half
        t_vecs = [
            [ht_v[sa + i, pl.ds(DIM + L * j, L)] for j in range(NBLK)]
            for i in range(2)
        ]

        def blk_body(db, accs, sa=sa, t_vecs=t_vecs):
            hv = [ht_v[sa + i, pl.ds(db * L, L)] for i in range(2)]
            a0e, a0o, a1e, a1o = accs
            for k in range(L):
                row = db * L + k
                m0 = mat_v[0, row, pl.ds(0, L)]
                p0 = m0 * t_vecs[0][0]
                p1 = m0 * t_vecs[1][0]
                m1 = mat_v[0, row, pl.ds(L, L)]
                q0 = m1 * t_vecs[0][1]
                q1 = m1 * t_vecs[1][1]
                for j in range(2, NBLK):
                    mj = mat_v[0, row, pl.ds(L * j, L)]
                    if j % 2 == 0:
                        p0 = p0 + mj * t_vecs[0][j]
                        p1 = p1 + mj * t_vecs[1][j]
                    else:
                        q0 = q0 + mj * t_vecs[0][j]
                        q1 = q1 + mj * t_vecs[1][j]
                d0 = p0 + q0
                d1 = p1 + q1
                if k % 2 == 0:
                    a0e = a0e + hv[0][k] * d0
                    a1e = a1e + hv[1][k] * d1
                else:
                    a0o = a0o + hv[0][k] * d0
                    a1o = a1o + hv[1][k] * d1
            return (a0e, a0o, a1e, a1o)

        zero = jnp.zeros((L,), jnp.float32)
        a0e, a0o, a1e, a1o = lax.fori_loop(0, NBLK, blk_body, (zero,) * 4)
        out_v[sa] = a0e + a0o
        out_v[sa + 1] = a1e + a1o
        return 0

    lax.fori_loop(0, S // 2, pass_body, 0)


def _sc_body(ht_hbm, iid_hbm, mid_hbm, cnt_hbm, table_hbm, out_hbm,
             iid_v, mid_v, cnt_v, ht_v, mat0_v, mat1_v, out_v,
             sem_h, sem0, sem1):
    cid = lax.axis_index("c")
    sid = lax.axis_index("s")
    wid = sid * 2 + cid

    pltpu.sync_copy(iid_hbm.at[wid], iid_v)
    pltpu.sync_copy(mid_hbm.at[wid], mid_v)
    pltpu.sync_copy(cnt_hbm, cnt_v)
    gpt = cnt_v[pl.ds(0, L)][0]
    qtr = gpt // 4

    pltpu.async_copy(table_hbm.at[mid_v.at[0]], mat0_v, sem0)
    pltpu.async_copy(table_hbm.at[mid_v.at[1]], mat1_v, sem1)

    for phase in range(2):
        pltpu.async_copy(ht_hbm.at[iid_v.at[phase]], ht_v, sem_h)
        pltpu.make_async_copy(ht_hbm.at[iid_v.at[phase]], ht_v, sem_h).wait()
        base_g = phase * (2 * qtr)

        def pair_body(p, _, base_g=base_g):
            g0 = base_g + 2 * p
            rb = (g0 - base_g) * S
            pltpu.make_async_copy(table_hbm.at[mid_v.at[g0]], mat0_v, sem0).wait()
            _compute_group(ht_v, mat0_v, out_v, rb)
            pltpu.async_copy(table_hbm.at[mid_v.at[g0 + 2]], mat0_v, sem0)

            pltpu.make_async_copy(table_hbm.at[mid_v.at[g0 + 1]], mat1_v, sem1).wait()
            _compute_group(ht_v, mat1_v, out_v, rb + S)
            pltpu.async_copy(table_hbm.at[mid_v.at[g0 + 3]], mat1_v, sem1)
            return 0

        lax.fori_loop(0, qtr, pair_body, 0)
        pltpu.sync_copy(out_v, out_hbm.at[wid, phase])

    pltpu.make_async_copy(table_hbm.at[mid_v.at[gpt]], mat0_v, sem0).wait()
    pltpu.make_async_copy(table_hbm.at[mid_v.at[gpt + 1]], mat1_v, sem1).wait()


@jax.jit
def _bilinear_sc(ht, iid, mid, cnt, table):
    mesh = plsc.VectorSubcoreMesh(core_axis_name="c", subcore_axis_name="s")
    fn = pl.kernel(
        _sc_body,
        out_type=jax.ShapeDtypeStruct((NW, 2, HALF, L), jnp.float32),
        mesh=mesh,
        scratch_types=[
            pltpu.VMEM((2, HALF), jnp.int32),
            pltpu.VMEM((MIDROWS, 1), jnp.int32),
            pltpu.VMEM((L,), jnp.int32),
            pltpu.VMEM((HALF, 2 * DIM), jnp.float32),
            pltpu.VMEM((1, DIM, DIM), jnp.float32),
            pltpu.VMEM((1, DIM, DIM), jnp.float32),
            pltpu.VMEM((HALF, L), jnp.float32),
            pltpu.SemaphoreType.DMA,
            pltpu.SemaphoreType.DMA,
            pltpu.SemaphoreType.DMA,
        ],
    )
    return fn(ht, iid, mid, cnt, table)


def kernel(heads_and_tails, relations, kernel):
    rel = relations[:, 0].astype(jnp.int32)

    packed = jnp.sort(rel * BATCH + jnp.arange(BATCH, dtype=jnp.int32))
    order = packed % BATCH
    srel = packed // BATCH
    counts = jnp.bincount(rel, length=NUM_RELATIONS)
    ng = (counts + (S - 1)) // S
    gbase = jnp.cumsum(ng) - ng
    total_g = jnp.sum(ng)
    gpt = ((total_g + 4 * NW - 1) // (4 * NW)) * 4
    segstart = jnp.cumsum(counts) - counts
    rank = jnp.arange(BATCH, dtype=jnp.int32) - segstart[srel]
    g_global = gbase[srel] + rank // S
    tile = g_global // gpt
    g_local = g_global % gpt
    half_g = gpt // 2
    phase = (g_local >= half_g).astype(jnp.int32)
    row_in_phase = (g_local - phase * half_g) * S + rank % S
    flat = tile * SLOTS + phase * HALF + row_in_phase

    iid = jnp.zeros((NW * SLOTS,), jnp.int32).at[flat].set(order.astype(jnp.int32))
    mid = jnp.zeros((NW * MIDROWS,), jnp.int32).at[tile * MIDROWS + g_local].set(srel)
    outpos = jnp.zeros((BATCH,), jnp.int32).at[order].set(flat)
    cnt = jnp.full((L,), gpt, jnp.int32)

    out16 = _bilinear_sc(
        heads_and_tails,
        iid.reshape(NW, 2, HALF),
        mid.reshape(NW, MIDROWS, 1),
        cnt,
        kernel,
    )
    out = jnp.sum(out16.reshape(NW * SLOTS, L), axis=1)[outpos]
    return out[:, None]


# speedup vs baseline: 3.4411x; 3.4411x over previous
import jax
import jax.numpy as jnp
from jax import lax
from jax.experimental import pallas as pl
from jax.experimental.pallas import tpu as pltpu
from jax.experimental.pallas import tpu_sc as plsc

NUM_RELATIONS = 1000
DIM = 128
BATCH = 4096
L = 16
NW = 32
BPW = BATCH // NW
CH = 2
NCHUNK = BPW // CH
NBLK = DIM // L


def _compute_item(ht_v, mat_v, out_v, ii, i):
    t_vecs = [ht_v[i, pl.ds(DIM + L * j, L)] for j in range(NBLK)]

    def blk_body(db, accs):
        hvec = ht_v[i, pl.ds(db * L, L)]
        ae, ao = accs
        for k in range(L):
            row = db * L + k
            p = mat_v[ii, row, pl.ds(0, L)] * t_vecs[0]
            q = mat_v[ii, row, pl.ds(L, L)] * t_vecs[1]
            for j in range(2, NBLK):
                mj = mat_v[ii, row, pl.ds(L * j, L)]
                if j % 2 == 0:
                    p = p + mj * t_vecs[j]
                else:
                    q = q + mj * t_vecs[j]
            d = p + q
            if k % 2 == 0:
                ae = ae + hvec[k] * d
            else:
                ao = ao + hvec[k] * d
        return (ae, ao)

    zero = jnp.zeros((L,), jnp.float32)
    ae, ao = lax.fori_loop(0, NBLK, blk_body, (zero, zero))
    out_v[i] = ae + ao


def _sc_body(ht_hbm, rel_hbm, table_hbm, out_hbm,
             idx_v, ht_v, mat0_v, mat1_v, out_v, sem0, sem1):
    cid = lax.axis_index("c")
    sid = lax.axis_index("s")
    wid = sid * 2 + cid
    base = wid * BPW

    pltpu.sync_copy(rel_hbm.at[wid], idx_v)
    pltpu.sync_copy(ht_hbm.at[pl.ds(base, BPW)], ht_v)

    pltpu.async_copy(table_hbm.at[idx_v.at[0]], mat0_v, sem0)
    pltpu.async_copy(table_hbm.at[idx_v.at[1]], mat1_v, sem1)

    def pair_body(p, _):
        c0 = 2 * p
        pltpu.make_async_copy(table_hbm.at[idx_v.at[c0]], mat0_v, sem0).wait()
        for ii in range(CH):
            _compute_item(ht_v, mat0_v, out_v, ii, c0 * CH + ii)
        pltpu.async_copy(table_hbm.at[idx_v.at[c0 + 2]], mat0_v, sem0)

        pltpu.make_async_copy(table_hbm.at[idx_v.at[c0 + 1]], mat1_v, sem1).wait()
        for ii in range(CH):
            _compute_item(ht_v, mat1_v, out_v, ii, (c0 + 1) * CH + ii)
        pltpu.async_copy(table_hbm.at[idx_v.at[c0 + 3]], mat1_v, sem1)
        return 0

    lax.fori_loop(0, NCHUNK // 2, pair_body, 0)

    pltpu.make_async_copy(table_hbm.at[idx_v.at[NCHUNK]], mat0_v, sem0).wait()
    pltpu.make_async_copy(table_hbm.at[idx_v.at[NCHUNK + 1]], mat1_v, sem1).wait()

    pltpu.sync_copy(out_v, out_hbm.at[pl.ds(base, BPW)])


@jax.jit
def _bilinear_sc(ht, rel, table):
    mesh = plsc.VectorSubcoreMesh(core_axis_name="c", subcore_axis_name="s")
    fn = pl.kernel(
        _sc_body,
        out_type=jax.ShapeDtypeStruct((BATCH, L), jnp.float32),
        mesh=mesh,
        scratch_types=[
            pltpu.VMEM((NCHUNK + 2, CH), jnp.int32),
            pltpu.VMEM((BPW, 2 * DIM), jnp.float32),
            pltpu.VMEM((CH, DIM, DIM), jnp.float32),
            pltpu.VMEM((CH, DIM, DIM), jnp.float32),
            pltpu.VMEM((BPW, L), jnp.float32),
            pltpu.SemaphoreType.DMA,
            pltpu.SemaphoreType.DMA,
        ],
    )
    return fn(ht, rel, table)


def kernel(heads_and_tails, relations, kernel):
    rel = relations[:, 0].astype(jnp.int32).reshape(NW, NCHUNK, CH)
    rel = jnp.pad(rel, ((0, 0), (0, 2), (0, 0)))
    out16 = _bilinear_sc(heads_and_tails, rel, kernel)
    return jnp.sum(out16, axis=1)[:, None]
